# baseline (speedup 1.0000x reference)
import functools

import jax
import jax.numpy as jnp
from jax import lax
from jax.experimental import pallas as pl
from jax.experimental.pallas import tpu as pltpu
from jax.experimental.pallas import tpu_sc as plsc

T = 16384
D_IN = 256
K = 8192
D_C = 32

TB = 256
NB = T // TB
NWIN = 2
SW = K // NWIN


def _xsq_reference_order(sq):
    g = sq[:, 0:8]
    for v in range(1, 4):
        g = g + sq[:, v * 8:(v + 1) * 8]
    h = g[:, 0:4] + g[:, 4:8]
    h = h[:, 0:2] + h[:, 2:4]
    return h[:, 0:1] + h[:, 1:2]


def _indices_body(x_ref, win_ref, e_ref, idx_ref):
    xb = x_ref[...].astype(jnp.bfloat16)
    wb = win_ref[...].astype(jnp.bfloat16)
    proj = lax.dot_general(xb, wb, (((1,), (1,)), ((), ())),
                           preferred_element_type=jnp.float32)
    x_sq = _xsq_reference_order(proj * proj)
    e = e_ref[...]
    e_sq = jnp.sum(e * e, axis=1)
    pb = proj.astype(jnp.bfloat16)
    eb = e.astype(jnp.bfloat16)
    cross = lax.dot_general(pb, eb, (((1,), (1,)), ((), ())),
                            preferred_element_type=jnp.float32)
    dist = (x_sq - 2.0 * cross) + e_sq[None, :]

    bar = jnp.full((TB,), jnp.inf, jnp.float32)
    bidx = jnp.zeros((TB,), jnp.int32)
    iota = lax.broadcasted_iota(jnp.int32, (TB, SW), 1)
    for w in range(NWIN):
        dw = dist[:, w * SW:(w + 1) * SW]
        vw = jnp.min(dw, axis=-1)
        iw = jnp.min(jnp.where(dw == vw[:, None], iota + w * SW, K), axis=-1)
        take = (vw < bar) | ((vw == bar) & (iw < bidx))
        bidx = jnp.where(take, iw, bidx)
        bar = jnp.where(take, vw, bar).astype(jnp.bfloat16).astype(jnp.float32)
    idx_ref[0, 0, :] = bidx


def _decode_body(e_ref, wout_ref, d_ref):
    eb = e_ref[...].astype(jnp.bfloat16)
    wb = wout_ref[...].astype(jnp.bfloat16)
    d_ref[...] = lax.dot_general(eb, wb, (((1,), (1,)), ((), ())),
                                 preferred_element_type=jnp.float32)


def _compute_indices(x_td, W_in, embeddings_kd):
    idx = pl.pallas_call(
        _indices_body,
        grid=(NB,),
        in_specs=[
            pl.BlockSpec((TB, D_IN), lambda i: (i, 0)),
            pl.BlockSpec((D_C, D_IN), lambda i: (0, 0)),
            pl.BlockSpec((K, D_C), lambda i: (0, 0)),
        ],
        out_specs=pl.BlockSpec((1, 1, TB), lambda i: (i, 0, 0)),
        out_shape=jax.ShapeDtypeStruct((NB, 1, TB), jnp.int32),
    )(x_td, W_in, embeddings_kd)
    return idx.reshape(T)


def _decoded_codebook(embeddings_kd, W_out):
    return pl.pallas_call(
        _decode_body,
        out_shape=jax.ShapeDtypeStruct((K, D_IN), jnp.float32),
    )(embeddings_kd, W_out)


def _sc_gather(d_kd, idx_t):
    info = plsc.get_sparse_core_info()
    nc, ns = info.num_cores, info.num_subcores
    nw = nc * ns
    rows_per_w = T // nw
    chunk = 256

    mesh = plsc.VectorSubcoreMesh(core_axis_name="c", subcore_axis_name="s")

    @functools.partial(
        pl.kernel,
        mesh=mesh,
        out_type=jax.ShapeDtypeStruct((T, D_IN), jnp.float32),
        scratch_types=[
            pltpu.VMEM((rows_per_w,), jnp.int32),
            pltpu.VMEM((chunk, D_IN), jnp.float32),
            pltpu.SemaphoreType.DMA,
        ],
    )
    def gather(d_hbm, idx_hbm, out_hbm, idx_v, rows_v, sem):
        wid = lax.axis_index("s") * nc + lax.axis_index("c")
        base = wid * rows_per_w
        pltpu.sync_copy(idx_hbm.at[pl.ds(base, rows_per_w)], idx_v)
        for c in range(rows_per_w // chunk):
            pltpu.async_copy(
                d_hbm.at[idx_v.at[pl.ds(c * chunk, chunk)]], rows_v, sem
            ).wait()
            pltpu.sync_copy(rows_v, out_hbm.at[pl.ds(base + c * chunk, chunk)])

    return gather(d_kd, idx_t)


def kernel(x_td, W_in, W_out, embeddings_kd):
    idx_t = _compute_indices(x_td, W_in, embeddings_kd)
    d_kd = _decoded_codebook(embeddings_kd, W_out)
    out_td = _sc_gather(d_kd, idx_t)
    return (out_td, idx_t)


# speedup vs baseline: 1.2286x; 1.2286x over previous
import functools

import jax
import jax.numpy as jnp
from jax import lax
from jax.experimental import pallas as pl
from jax.experimental.pallas import tpu as pltpu
from jax.experimental.pallas import tpu_sc as plsc

T = 16384
D_IN = 256
K = 8192
D_C = 32

TB = 512
NB = T // TB
NWIN = 2
SW = K // NWIN


def _xsq_reference_order(sq):
    g = sq[:, 0:8]
    for v in range(1, 4):
        g = g + sq[:, v * 8:(v + 1) * 8]
    h = g[:, 0:4] + g[:, 4:8]
    h = h[:, 0:2] + h[:, 2:4]
    return h[:, 0:1] + h[:, 1:2]


def _indices_body(x_ref, win_ref, e_ref, idx_ref):
    xb = x_ref[...].astype(jnp.bfloat16)
    wb = win_ref[...].astype(jnp.bfloat16)
    proj = lax.dot_general(xb, wb, (((1,), (1,)), ((), ())),
                           preferred_element_type=jnp.float32)
    x_sq = _xsq_reference_order(proj * proj)
    e = e_ref[...]
    e_sq = jnp.sum(e * e, axis=1)
    pb = (proj.astype(jnp.bfloat16)) * jnp.bfloat16(-2.0)
    eb = e.astype(jnp.bfloat16)
    neg2cross = lax.dot_general(pb, eb, (((1,), (1,)), ((), ())),
                                preferred_element_type=jnp.float32)
    dist = (x_sq + neg2cross) + e_sq[None, :]

    bar = jnp.full((TB,), jnp.inf, jnp.float32)
    bidx = jnp.zeros((TB,), jnp.int32)
    iota = lax.broadcasted_iota(jnp.int32, (TB, SW), 1)
    for w in range(NWIN):
        dw = dist[:, w * SW:(w + 1) * SW]
        vw = jnp.min(dw, axis=-1)
        iw = jnp.min(jnp.where(dw == vw[:, None], iota + w * SW, K), axis=-1)
        take = (vw < bar) | ((vw == bar) & (iw < bidx))
        bidx = jnp.where(take, iw, bidx)
        bar = jnp.where(take, vw, bar).astype(jnp.bfloat16).astype(jnp.float32)
    idx_ref[0, 0, :] = bidx


def _decode_body(e_ref, wout_ref, d_ref):
    eb = e_ref[...].astype(jnp.bfloat16)
    wb = wout_ref[...].astype(jnp.bfloat16)
    d_ref[...] = lax.dot_general(eb, wb, (((1,), (1,)), ((), ())),
                                 preferred_element_type=jnp.float32)


def _compute_indices(x_td, W_in, embeddings_kd):
    idx = pl.pallas_call(
        _indices_body,
        grid=(NB,),
        in_specs=[
            pl.BlockSpec((TB, D_IN), lambda i: (i, 0)),
            pl.BlockSpec((D_C, D_IN), lambda i: (0, 0)),
            pl.BlockSpec((K, D_C), lambda i: (0, 0)),
        ],
        out_specs=pl.BlockSpec((1, 1, TB), lambda i: (i, 0, 0)),
        out_shape=jax.ShapeDtypeStruct((NB, 1, TB), jnp.int32),
    )(x_td, W_in, embeddings_kd)
    return idx.reshape(T)


def _decoded_codebook(embeddings_kd, W_out):
    return pl.pallas_call(
        _decode_body,
        out_shape=jax.ShapeDtypeStruct((K, D_IN), jnp.float32),
    )(embeddings_kd, W_out)


def _sc_gather(d_kd, idx_t):
    info = plsc.get_sparse_core_info()
    nc, ns = info.num_cores, info.num_subcores
    nw = nc * ns
    rows_per_w = T // nw
    chunk = 256

    mesh = plsc.VectorSubcoreMesh(core_axis_name="c", subcore_axis_name="s")

    @functools.partial(
        pl.kernel,
        mesh=mesh,
        out_type=jax.ShapeDtypeStruct((T, D_IN), jnp.float32),
        scratch_types=[
            pltpu.VMEM((rows_per_w,), jnp.int32),
            pltpu.VMEM((chunk, D_IN), jnp.float32),
            pltpu.SemaphoreType.DMA,
        ],
    )
    def gather(d_hbm, idx_hbm, out_hbm, idx_v, rows_v, sem):
        wid = lax.axis_index("s") * nc + lax.axis_index("c")
        base = wid * rows_per_w
        pltpu.sync_copy(idx_hbm.at[pl.ds(base, rows_per_w)], idx_v)
        for c in range(rows_per_w // chunk):
            pltpu.async_copy(
                d_hbm.at[idx_v.at[pl.ds(c * chunk, chunk)]], rows_v, sem
            ).wait()
            pltpu.sync_copy(rows_v, out_hbm.at[pl.ds(base + c * chunk, chunk)])

    return gather(d_kd, idx_t)


def kernel(x_td, W_in, W_out, embeddings_kd):
    idx_t = _compute_indices(x_td, W_in, embeddings_kd)
    d_kd = _decoded_codebook(embeddings_kd, W_out)
    out_td = _sc_gather(d_kd, idx_t)
    return (out_td, idx_t)
